# baseline (speedup 1.0000x reference)
import functools

import jax
import jax.numpy as jnp
from jax import lax
from jax.experimental import pallas as pl
from jax.experimental.pallas import tpu as pltpu
from jax.experimental.pallas import tpu_sc as plsc

N = 50000
E = 800000
G = 64
H = 64

NC = 2
NS = 16
NW = NC * NS
LANES = 128

CHUNKS = 196
EPT = CHUNKS * LANES
EPAD = NW * EPT
VPAD = 50176
NACC = 51200
VCH = VPAD // NS
ZCH = NACC // NS
NROW = 49
NCOL = 1024


_MESH = dict(core_axis_name="c", subcore_axis_name="s",
             num_cores=NC, num_subcores=NS)


def _zero_fill(zb):
    def zstep(i, _):
        zb[pl.ds(i * 16, 16)] = jnp.zeros((16,), jnp.float32)
        return 0
    lax.fori_loop(0, ZCH // 16, zstep, 0)


def _edge_stream_loop(gather, num_vals, src_v, dst_v, vals_v, vshared, acc,
                      sem_g, sem_s):
    if not gather:
        def frow(i, _):
            def fcol(j, _):
                vals_v[0][i, pl.ds(j * 16, 16)] = jnp.ones((16,), jnp.float32)
                return 0
            lax.fori_loop(0, LANES // 16, fcol, 0)
            return 0
        lax.fori_loop(0, CHUNKS, frow, 0)

    U = 7 if num_vals == 1 else 4

    def group(g, _):
        base = g * U
        if gather:
            descs = [pltpu.async_copy(vshared[k].at[src_v.at[base + u]],
                                      vals_v[k].at[base + u], sem_g)
                     for u in range(U) for k in range(num_vals)]
            for d in descs:
                d.wait()
        for u in range(U):
            for k in range(num_vals):
                pltpu.async_copy(vals_v[k].at[base + u],
                                 acc[k].at[dst_v.at[base + u]], sem_s,
                                 add=True)
        return 0
    lax.fori_loop(0, CHUNKS // U, group, 0)

    def drain(j, _):
        for k in range(num_vals):
            pltpu.make_async_copy(vals_v[k].at[0],
                                  acc[k].at[dst_v.at[0]], sem_s).wait()
        return 0
    lax.fori_loop(0, CHUNKS, drain, 0)


def _newton_rsqrt(x):
    ih = jnp.int32(0x5F3759DF) - lax.shift_right_logical(
        lax.bitcast_convert_type(x, jnp.int32), 1)
    y = lax.bitcast_convert_type(ih, jnp.float32)
    for _ in range(3):
        y = y * (1.5 - 0.5 * x * y * y)
    return y


def _make_k1():
    mesh = plsc.VectorSubcoreMesh(**_MESH)
    scratch = [
        pltpu.VMEM((VCH,), jnp.float32),
        pltpu.VMEM((VCH,), jnp.float32),
        pltpu.VMEM((VCH,), jnp.float32),
        pltpu.VMEM((CHUNKS, LANES), jnp.int32),
        pltpu.VMEM((CHUNKS, LANES), jnp.int32),
        pltpu.VMEM((CHUNKS, LANES), jnp.float32),
        pltpu.VMEM((ZCH,), jnp.float32),
        pltpu.VMEM((ZCH,), jnp.float32),
        pltpu.VMEM_SHARED((VPAD,), jnp.float32),
        pltpu.VMEM_SHARED((NACC,), jnp.float32),
        (pltpu.SemaphoreType.DMA, pltpu.SemaphoreType.DMA),
    ]
    out_type = (jax.ShapeDtypeStruct((NC * NACC,), jnp.float32),
                jax.ShapeDtypeStruct((VPAD,), jnp.float32),
                jax.ShapeDtypeStruct((VPAD,), jnp.float32))

    @functools.partial(pl.kernel, out_type=out_type, mesh=mesh,
                       scratch_types=scratch)
    def k1(degp_hbm, x_hbm, src_hbm, dst_hbm, accx_hbm, dinv_hbm, xhat_hbm,
           d0b, d1b, xb, src_v, dst_v, vals_v, zb, stg, vsh, acc, sems):
        sem_g, sem_s = sems
        cid = lax.axis_index("c")
        sid = lax.axis_index("s")
        w = cid * NS + sid
        sl = pl.ds(sid * VCH, VCH)

        _zero_fill(zb)
        pltpu.sync_copy(zb, acc.at[pl.ds(sid * ZCH, ZCH)])

        pltpu.sync_copy(degp_hbm.at[pl.ds(sid * VCH, VCH)], d0b)
        pltpu.sync_copy(degp_hbm.at[pl.ds(NACC + sid * VCH, VCH)], d1b)
        pltpu.sync_copy(x_hbm.at[sl], xb)

        def dstep(i, _):
            v = pl.ds(i * 16, 16)
            deg = d0b[v] + d1b[v] + 1.0
            y = _newton_rsqrt(deg)
            d0b[v] = y
            d1b[v] = y * xb[v]
            return 0
        lax.fori_loop(0, VCH // 16, dstep, 0)

        pltpu.sync_copy(d0b, dinv_hbm.at[sl])
        pltpu.sync_copy(d1b, xhat_hbm.at[sl])
        pltpu.sync_copy(d1b, vsh.at[sl])
        plsc.subcore_barrier()

        pltpu.sync_copy(src_hbm.at[w], src_v)
        pltpu.sync_copy(dst_hbm.at[w], dst_v)
        _edge_stream_loop(True, 1, src_v, dst_v, [vals_v], [vsh], [acc],
                          sem_g, sem_s)

        plsc.subcore_barrier()
        pltpu.sync_copy(acc.at[pl.ds(sid * ZCH, ZCH)], stg)
        pltpu.sync_copy(stg, accx_hbm.at[pl.ds(cid * NACC + sid * ZCH, ZCH)])

    return k1


def _make_k2():
    mesh = plsc.VectorSubcoreMesh(**_MESH)
    scratch = [
        pltpu.VMEM((VCH,), jnp.float32),
        pltpu.VMEM((VCH,), jnp.float32),
        pltpu.VMEM((VCH,), jnp.float32),
        pltpu.VMEM((VCH,), jnp.float32),
        pltpu.VMEM((CHUNKS, LANES), jnp.int32),
        pltpu.VMEM((CHUNKS, LANES), jnp.int32),
        pltpu.VMEM((CHUNKS, LANES), jnp.float32),
        pltpu.VMEM((CHUNKS, LANES), jnp.float32),
        pltpu.VMEM((ZCH,), jnp.float32),
        pltpu.VMEM((ZCH,), jnp.float32),
        pltpu.VMEM_SHARED((NACC,), jnp.float32),
        pltpu.VMEM_SHARED((NACC,), jnp.float32),
        (pltpu.SemaphoreType.DMA, pltpu.SemaphoreType.DMA),
    ]
    out_type = (jax.ShapeDtypeStruct((NC * 2 * NACC,), jnp.float32),
                jax.ShapeDtypeStruct((VPAD,), jnp.float32),
                jax.ShapeDtypeStruct((VPAD,), jnp.float32))

    @functools.partial(pl.kernel, out_type=out_type, mesh=mesh,
                       scratch_types=scratch)
    def k2(accx_hbm, dinv_hbm, xhat_hbm, src_hbm, dst_hbm,
           accac_hbm, ah_hbm, ch_hbm,
           a0b, a1b, xb, db, src_v, dst_v, va, vc, zb, stg,
           acc0, acc1, sems):
        sem_g, sem_s = sems
        cid = lax.axis_index("c")
        sid = lax.axis_index("s")
        w = cid * NS + sid
        sl = pl.ds(sid * VCH, VCH)

        _zero_fill(zb)
        pltpu.sync_copy(zb, acc0.at[pl.ds(sid * ZCH, ZCH)])
        pltpu.sync_copy(zb, acc1.at[pl.ds(sid * ZCH, ZCH)])

        pltpu.sync_copy(accx_hbm.at[pl.ds(sid * VCH, VCH)], a0b)
        pltpu.sync_copy(accx_hbm.at[pl.ds(NACC + sid * VCH, VCH)], a1b)
        pltpu.sync_copy(xhat_hbm.at[sl], xb)
        pltpu.sync_copy(dinv_hbm.at[sl], db)

        def dstep(i, _):
            v = pl.ds(i * 16, 16)
            dv = db[v]
            s = dv * (a0b[v] + a1b[v] + xb[v])
            a0b[v] = dv * jnp.maximum(s, 0.0)
            a1b[v] = dv * jnp.minimum(s, 0.0)
            return 0
        lax.fori_loop(0, VCH // 16, dstep, 0)

        pltpu.sync_copy(a0b, ah_hbm.at[sl])
        pltpu.sync_copy(a1b, ch_hbm.at[sl])
        plsc.subcore_barrier()

        pltpu.sync_copy(src_hbm.at[w], src_v)
        pltpu.sync_copy(dst_hbm.at[w], dst_v)
        _edge_stream_loop(True, 2, src_v, dst_v, [va, vc], [ah_hbm, ch_hbm],
                          [acc0, acc1], sem_g, sem_s)

        plsc.subcore_barrier()
        for k, a in enumerate((acc0, acc1)):
            pltpu.sync_copy(a.at[pl.ds(sid * ZCH, ZCH)], stg)
            pltpu.sync_copy(
                stg,
                accac_hbm.at[pl.ds((cid * 2 + k) * NACC + sid * ZCH, ZCH)])

    return k2


def _make_edge_pass(num_vals, gather):
    mesh = plsc.VectorSubcoreMesh(
        core_axis_name="c", subcore_axis_name="s",
        num_cores=NC, num_subcores=NS)

    scratch = []
    if gather:
        scratch.append(pltpu.VMEM((CHUNKS, LANES), jnp.int32))
    scratch.append(pltpu.VMEM((CHUNKS, LANES), jnp.int32))
    for _ in range(num_vals):
        scratch.append(pltpu.VMEM((CHUNKS, LANES), jnp.float32))
    scratch.append(pltpu.VMEM((ZCH,), jnp.float32))
    scratch.append(pltpu.VMEM((ZCH,), jnp.float32))
    if gather:
        for _ in range(num_vals):
            scratch.append(pltpu.VMEM_SHARED((VPAD,), jnp.float32))
    for _ in range(num_vals):
        scratch.append(pltpu.VMEM_SHARED((NACC,), jnp.float32))
    scratch.append((pltpu.SemaphoreType.DMA, pltpu.SemaphoreType.DMA))

    out_type = jax.ShapeDtypeStruct((NC, num_vals, NACC), jnp.float32)

    @functools.partial(pl.kernel, out_type=out_type, mesh=mesh,
                       scratch_types=scratch)
    def edge_pass(*refs):
        it = iter(refs)
        vals_hbm = [next(it) for _ in range(num_vals)] if gather else []
        src_hbm = next(it) if gather else None
        dst_hbm = next(it)
        out_hbm = next(it)
        src_v = next(it) if gather else None
        dst_v = next(it)
        vals_v = [next(it) for _ in range(num_vals)]
        zb = next(it)
        stg = next(it)
        vshared = [next(it) for _ in range(num_vals)] if gather else []
        acc = [next(it) for _ in range(num_vals)]
        sem = next(it)

        cid = lax.axis_index("c")
        sid = lax.axis_index("s")
        w = cid * NS + sid

        def zstep(i, _):
            zb[pl.ds(i * 16, 16)] = jnp.zeros((16,), jnp.float32)
            return 0
        lax.fori_loop(0, ZCH // 16, zstep, 0)
        for k in range(num_vals):
            pltpu.sync_copy(zb, acc[k].at[pl.ds(sid * ZCH, ZCH)])

        if gather:
            for k in range(num_vals):
                pltpu.sync_copy(vals_hbm[k].at[pl.ds(sid * VCH, VCH)],
                                stg.at[pl.ds(0, VCH)])
                pltpu.sync_copy(stg.at[pl.ds(0, VCH)],
                                vshared[k].at[pl.ds(sid * VCH, VCH)])
        plsc.subcore_barrier()

        if gather:
            pltpu.sync_copy(src_hbm.at[w], src_v)
        pltpu.sync_copy(dst_hbm.at[w], dst_v)

        if not gather:
            def frow(i, _):
                def fcol(j, _):
                    vals_v[0][i, pl.ds(j * 16, 16)] = jnp.ones(
                        (16,), jnp.float32)
                    return 0
                lax.fori_loop(0, LANES // 16, fcol, 0)
                return 0
            lax.fori_loop(0, CHUNKS, frow, 0)

        U = 7 if num_vals == 1 else 4
        sem_g, sem_s = sem

        def group(g, _):
            base = g * U
            if gather:
                descs = [pltpu.async_copy(vshared[k].at[src_v.at[base + u]],
                                          vals_v[k].at[base + u], sem_g)
                         for u in range(U) for k in range(num_vals)]
                for d in descs:
                    d.wait()
            for u in range(U):
                for k in range(num_vals):
                    pltpu.async_copy(vals_v[k].at[base + u],
                                     acc[k].at[dst_v.at[base + u]], sem_s,
                                     add=True)
            return 0
        lax.fori_loop(0, CHUNKS // U, group, 0)

        def drain(j, _):
            for k in range(num_vals):
                pltpu.make_async_copy(vals_v[k].at[0],
                                      acc[k].at[dst_v.at[0]], sem_s).wait()
            return 0
        lax.fori_loop(0, CHUNKS, drain, 0)

        plsc.subcore_barrier()
        for k in range(num_vals):
            pltpu.sync_copy(acc[k].at[pl.ds(sid * ZCH, ZCH)], stg)
            pltpu.sync_copy(stg, out_hbm.at[cid, k, pl.ds(sid * ZCH, ZCH)])

    return edge_pass


_deg_pass = _make_edge_pass(1, gather=False)
_prop1_pass = _make_edge_pass(1, gather=True)
_k1_pass = _make_k1()
_k2_pass = _make_k2()


def _tc2(dinv, ahat, chat, accac, W1, W2, b2, W3, Wl):
    def body(dinv_ref, ahat_ref, chat_ref, acc_ref, w1_ref, w2_ref, b2_ref,
             w3_ref, wl_ref, vhat_ref):
        w1 = w1_ref[...]
        up = jnp.maximum(w1, 0.0) @ w2_ref[...]
        um = jnp.minimum(w1, 0.0) @ w2_ref[...]
        gv = w3_ref[...] @ wl_ref[...]
        b2v = b2_ref[...]
        dinv = dinv_ref[...]
        pa = dinv * (acc_ref[0, 0] + acc_ref[1, 0] + ahat_ref[...])
        pc = dinv * (acc_ref[0, 1] + acc_ref[1, 1] + chat_ref[...])
        v = jnp.zeros_like(pa)
        for j in range(H):
            v = v + jnp.maximum(pa * up[0, j] + pc * um[0, j] + b2v[0, j],
                                0.0) * gv[j, 0]
        vhat_ref[...] = dinv * v

    full = lambda s: pl.BlockSpec(s, lambda i: (0,) * len(s))
    return pl.pallas_call(
        body,
        grid=(NROW,),
        in_specs=[
            pl.BlockSpec((8, 128), lambda i: (i, 0)),
            pl.BlockSpec((8, 128), lambda i: (i, 0)),
            pl.BlockSpec((8, 128), lambda i: (i, 0)),
            pl.BlockSpec((2, 2, 8, 128), lambda i: (0, 0, i, 0)),
            full((1, H)), full((H, H)), full((1, H)), full((H, H)),
            full((H, 1)),
        ],
        out_specs=pl.BlockSpec((8, 128), lambda i: (i, 0)),
        out_shape=jax.ShapeDtypeStruct((NROW * 8, 128), jnp.float32),
    )(dinv, ahat, chat, accac, W1, W2, b2, W3, Wl)


def _tc3(dinv, vhat, accv, batch2, b3, Wl, bl):
    def body(dinv_ref, vhat_ref, acc_ref, batch_ref, b3_ref, wl_ref, bl_ref,
             out_ref):
        r = dinv_ref[...] * (acc_ref[0] + acc_ref[1] + vhat_ref[...])
        gids = lax.broadcasted_iota(jnp.int32, (G, 1), 0)
        sums = jnp.zeros((G, 1), jnp.float32)
        cnts = jnp.zeros((G, 1), jnp.float32)
        for i in range(NROW):
            oh = (batch_ref[i:i + 1, :] == gids).astype(jnp.float32)
            sums = sums + lax.dot_general(
                oh, r[i:i + 1, :], (((1,), (1,)), ((), ())))
            cnts = cnts + jnp.sum(oh, axis=1, keepdims=True)
        cst = b3_ref[...] @ wl_ref[...] + bl_ref[...]
        out_ref[...] = sums / jnp.maximum(cnts, 1.0) + cst

    return pl.pallas_call(
        body,
        out_shape=jax.ShapeDtypeStruct((G, 1), jnp.float32),
    )(dinv, vhat, accv, batch2, b3, Wl, bl)


def kernel(x, edge_index, batch, W1, b1, W2, b2, W3, b3, Wl, bl):
    src = edge_index[0]
    dst = edge_index[1]

    npad = EPAD - E
    src_p = jnp.concatenate([src, jnp.zeros((npad,), jnp.int32)])
    trash = VPAD + (jnp.arange(npad, dtype=jnp.int32) % (NACC - VPAD))
    dst_p = jnp.concatenate([dst, trash])
    src3 = src_p.reshape(NW, CHUNKS, LANES)
    dst3 = dst_p.reshape(NW, CHUNKS, LANES)

    xv = jnp.pad(x[:, 0], (0, VPAD - N))
    batch_p = jnp.pad(batch, (0, VPAD - N), constant_values=1 << 20)
    batch2 = batch_p.reshape(NROW, NCOL)

    degp = _deg_pass(dst3)

    accx, dinvc, xhatc = _k1_pass(degp[:, 0].reshape(-1), xv, src3, dst3)

    accac, ahc, chc = _k2_pass(accx, dinvc, xhatc, src3, dst3)

    dinv2 = dinvc.reshape(NROW * 8, 128)
    accac2 = accac.reshape(2, 2, NACC)[:, :, :VPAD].reshape(
        2, 2, NROW * 8, 128)

    b2r = b2.reshape(1, H)
    vhat2 = _tc2(dinv2, ahc.reshape(NROW * 8, 128),
                 chc.reshape(NROW * 8, 128), accac2, W1, W2, b2r,
                 W3, Wl)

    accv = _prop1_pass(vhat2.reshape(VPAD), src3, dst3)
    accv2 = accv[:, 0, :VPAD].reshape(2, NROW, NCOL)

    return _tc3(dinv2.reshape(NROW, NCOL), vhat2.reshape(NROW, NCOL),
                accv2, batch2, b3.reshape(1, H), Wl, bl.reshape(1, 1))


# speedup vs baseline: 1.2321x; 1.2321x over previous
import functools

import jax
import jax.numpy as jnp
from jax import lax
from jax.experimental import pallas as pl
from jax.experimental.pallas import tpu as pltpu
from jax.experimental.pallas import tpu_sc as plsc

N = 50000
E = 800000
G = 64
H = 64

NC = 2
NS = 16
NW = NC * NS
LANES = 128

CHUNKS = 196
EPT = CHUNKS * LANES
EPAD = NW * EPT
VPAD = 50176
NACC = 51200
VCH = VPAD // NS
ZCH = NACC // NS
NROW = 49
NCOL = 1024


_MESH = dict(core_axis_name="c", subcore_axis_name="s",
             num_cores=NC, num_subcores=NS)


def _zero_fill(zb):
    def zstep(i, _):
        zb[pl.ds(i * 16, 16)] = jnp.zeros((16,), jnp.float32)
        return 0
    lax.fori_loop(0, ZCH // 16, zstep, 0)


def _edge_stream_loop(gather, num_vals, src_v, dst_v, vals_v, vshared, acc,
                      sem_g, sem_s):
    if not gather:
        def frow(i, _):
            def fcol(j, _):
                vals_v[0][i, pl.ds(j * 16, 16)] = jnp.ones((16,), jnp.float32)
                return 0
            lax.fori_loop(0, LANES // 16, fcol, 0)
            return 0
        lax.fori_loop(0, CHUNKS, frow, 0)

    U = 7 if num_vals == 1 else 4

    def group(g, _):
        base = g * U
        if gather:
            descs = [pltpu.async_copy(vshared[k].at[src_v.at[base + u]],
                                      vals_v[k].at[base + u], sem_g)
                     for u in range(U) for k in range(num_vals)]
            for d in descs:
                d.wait()
        for u in range(U):
            for k in range(num_vals):
                pltpu.async_copy(vals_v[k].at[base + u],
                                 acc[k].at[dst_v.at[base + u]], sem_s,
                                 add=True)
        return 0
    lax.fori_loop(0, CHUNKS // U, group, 0)

    def drain(j, _):
        for k in range(num_vals):
            pltpu.make_async_copy(vals_v[k].at[0],
                                  acc[k].at[dst_v.at[0]], sem_s).wait()
        return 0
    lax.fori_loop(0, CHUNKS, drain, 0)


def _newton_rsqrt(x):
    ih = jnp.int32(0x5F3759DF) - lax.shift_right_logical(
        lax.bitcast_convert_type(x, jnp.int32), 1)
    y = lax.bitcast_convert_type(ih, jnp.float32)
    for _ in range(3):
        y = y * (1.5 - 0.5 * x * y * y)
    return y


def _ones_fill(vals_v):
    def frow(i, _):
        def fcol(j, _):
            vals_v[i, pl.ds(j * 16, 16)] = jnp.ones((16,), jnp.float32)
            return 0
        lax.fori_loop(0, LANES // 16, fcol, 0)
        return 0
    lax.fori_loop(0, CHUNKS, frow, 0)


def _make_k1():
    mesh = plsc.VectorSubcoreMesh(**_MESH)
    scratch = [
        pltpu.VMEM((VCH,), jnp.float32),
        pltpu.VMEM((VCH,), jnp.float32),
        pltpu.VMEM((VCH,), jnp.float32),
        pltpu.VMEM((CHUNKS, LANES), jnp.int32),
        pltpu.VMEM((CHUNKS, LANES), jnp.int32),
        pltpu.VMEM((CHUNKS, LANES), jnp.float32),
        pltpu.VMEM((ZCH,), jnp.float32),
        pltpu.VMEM_SHARED((VPAD,), jnp.float32),
        pltpu.VMEM_SHARED((NACC,), jnp.float32),
        pltpu.VMEM_SHARED((NACC,), jnp.float32),
        (pltpu.SemaphoreType.DMA, pltpu.SemaphoreType.DMA),
    ]
    out_type = (jax.ShapeDtypeStruct((NC * NACC,), jnp.float32),
                jax.ShapeDtypeStruct((VPAD,), jnp.float32),
                jax.ShapeDtypeStruct((VPAD,), jnp.float32))

    @functools.partial(pl.kernel, out_type=out_type, mesh=mesh,
                       scratch_types=scratch)
    def k1(x_hbm, src_hbm, dst_hbm, accx_hbm, dinv_hbm, xhat_hbm,
           d0b, d1b, xb, src_v, dst_v, vals_v, zb, vsh, dacc, acc, sems):
        sem_g, sem_s = sems
        cid = lax.axis_index("c")
        sid = lax.axis_index("s")
        w = cid * NS + sid
        sl = pl.ds(sid * VCH, VCH)

        _zero_fill(zb)
        pltpu.sync_copy(zb, dacc.at[pl.ds(sid * ZCH, ZCH)])
        pltpu.sync_copy(zb, acc.at[pl.ds(sid * ZCH, ZCH)])
        _ones_fill(vals_v)
        pltpu.sync_copy(x_hbm.at[sl], xb)
        plsc.subcore_barrier()

        pltpu.sync_copy(dst_hbm.at[2 * sid], src_v)
        pltpu.sync_copy(dst_hbm.at[2 * sid + 1], dst_v)

        def dgroup(g, _):
            base = g * 7
            for u in range(7):
                pltpu.async_copy(vals_v.at[base + u],
                                 dacc.at[src_v.at[base + u]], sem_s,
                                 add=True)
                pltpu.async_copy(vals_v.at[base + u],
                                 dacc.at[dst_v.at[base + u]], sem_s,
                                 add=True)
            return 0
        lax.fori_loop(0, CHUNKS // 7, dgroup, 0)

        def ddrain(j, _):
            pltpu.make_async_copy(vals_v.at[0], dacc.at[src_v.at[0]],
                                  sem_s).wait()
            return 0
        lax.fori_loop(0, 2 * CHUNKS, ddrain, 0)
        plsc.subcore_barrier()

        pltpu.sync_copy(dacc.at[sl], d0b)

        def dstep(i, _):
            v = pl.ds(i * 16, 16)
            deg = d0b[v] + 1.0
            y = _newton_rsqrt(deg)
            d0b[v] = y
            d1b[v] = y * xb[v]
            return 0
        lax.fori_loop(0, VCH // 16, dstep, 0)

        pltpu.sync_copy(d0b, dinv_hbm.at[sl])
        pltpu.sync_copy(d1b, xhat_hbm.at[sl])
        pltpu.sync_copy(d1b, vsh.at[sl])
        plsc.subcore_barrier()

        pltpu.sync_copy(src_hbm.at[w], src_v)
        pltpu.sync_copy(dst_hbm.at[w], dst_v)
        _edge_stream_loop(True, 1, src_v, dst_v, [vals_v], [vsh], [acc],
                          sem_g, sem_s)

        plsc.subcore_barrier()
        pltpu.sync_copy(acc.at[pl.ds(sid * ZCH, ZCH)], zb)
        pltpu.sync_copy(zb, accx_hbm.at[pl.ds(cid * NACC + sid * ZCH, ZCH)])

    return k1


def _make_k2():
    mesh = plsc.VectorSubcoreMesh(**_MESH)
    scratch = [
        pltpu.VMEM((VCH,), jnp.float32),
        pltpu.VMEM((VCH,), jnp.float32),
        pltpu.VMEM((VCH,), jnp.float32),
        pltpu.VMEM((VCH,), jnp.float32),
        pltpu.VMEM((CHUNKS, LANES), jnp.int32),
        pltpu.VMEM((CHUNKS, LANES), jnp.int32),
        pltpu.VMEM((CHUNKS, LANES), jnp.float32),
        pltpu.VMEM((CHUNKS, LANES), jnp.float32),
        pltpu.VMEM((ZCH,), jnp.float32),
        pltpu.VMEM_SHARED((VPAD,), jnp.float32),
        pltpu.VMEM_SHARED((VPAD,), jnp.float32),
        pltpu.VMEM_SHARED((NACC,), jnp.float32),
        pltpu.VMEM_SHARED((NACC,), jnp.float32),
        (pltpu.SemaphoreType.DMA, pltpu.SemaphoreType.DMA),
    ]
    out_type = (jax.ShapeDtypeStruct((NC * 2 * NACC,), jnp.float32),
                jax.ShapeDtypeStruct((VPAD,), jnp.float32),
                jax.ShapeDtypeStruct((VPAD,), jnp.float32))

    @functools.partial(pl.kernel, out_type=out_type, mesh=mesh,
                       scratch_types=scratch)
    def k2(accx_hbm, dinv_hbm, xhat_hbm, src_hbm, dst_hbm,
           accac_hbm, ah_hbm, ch_hbm,
           a0b, a1b, xb, db, src_v, dst_v, va, vc, zb,
           vsh0, vsh1, acc0, acc1, sems):
        sem_g, sem_s = sems
        cid = lax.axis_index("c")
        sid = lax.axis_index("s")
        w = cid * NS + sid
        sl = pl.ds(sid * VCH, VCH)

        _zero_fill(zb)
        pltpu.sync_copy(zb, acc0.at[pl.ds(sid * ZCH, ZCH)])
        pltpu.sync_copy(zb, acc1.at[pl.ds(sid * ZCH, ZCH)])

        pltpu.sync_copy(accx_hbm.at[pl.ds(sid * VCH, VCH)], a0b)
        pltpu.sync_copy(accx_hbm.at[pl.ds(NACC + sid * VCH, VCH)], a1b)
        pltpu.sync_copy(xhat_hbm.at[sl], xb)
        pltpu.sync_copy(dinv_hbm.at[sl], db)

        def dstep(i, _):
            v = pl.ds(i * 16, 16)
            dv = db[v]
            s = dv * (a0b[v] + a1b[v] + xb[v])
            a0b[v] = dv * jnp.maximum(s, 0.0)
            a1b[v] = dv * jnp.minimum(s, 0.0)
            return 0
        lax.fori_loop(0, VCH // 16, dstep, 0)

        pltpu.sync_copy(a0b, ah_hbm.at[sl])
        pltpu.sync_copy(a1b, ch_hbm.at[sl])
        pltpu.sync_copy(a0b, vsh0.at[sl])
        pltpu.sync_copy(a1b, vsh1.at[sl])
        plsc.subcore_barrier()

        pltpu.sync_copy(src_hbm.at[w], src_v)
        pltpu.sync_copy(dst_hbm.at[w], dst_v)
        _edge_stream_loop(True, 2, src_v, dst_v, [va, vc], [vsh0, vsh1],
                          [acc0, acc1], sem_g, sem_s)

        plsc.subcore_barrier()
        for k, a in enumerate((acc0, acc1)):
            pltpu.sync_copy(a.at[pl.ds(sid * ZCH, ZCH)], zb)
            pltpu.sync_copy(
                zb,
                accac_hbm.at[pl.ds((cid * 2 + k) * NACC + sid * ZCH, ZCH)])

    return k2


def _make_edge_pass(num_vals, gather):
    mesh = plsc.VectorSubcoreMesh(
        core_axis_name="c", subcore_axis_name="s",
        num_cores=NC, num_subcores=NS)

    scratch = []
    if gather:
        scratch.append(pltpu.VMEM((CHUNKS, LANES), jnp.int32))
    scratch.append(pltpu.VMEM((CHUNKS, LANES), jnp.int32))
    for _ in range(num_vals):
        scratch.append(pltpu.VMEM((CHUNKS, LANES), jnp.float32))
    scratch.append(pltpu.VMEM((ZCH,), jnp.float32))
    scratch.append(pltpu.VMEM((ZCH,), jnp.float32))
    if gather:
        for _ in range(num_vals):
            scratch.append(pltpu.VMEM_SHARED((VPAD,), jnp.float32))
    for _ in range(num_vals):
        scratch.append(pltpu.VMEM_SHARED((NACC,), jnp.float32))
    scratch.append((pltpu.SemaphoreType.DMA, pltpu.SemaphoreType.DMA))

    out_type = jax.ShapeDtypeStruct((NC, num_vals, NACC), jnp.float32)

    @functools.partial(pl.kernel, out_type=out_type, mesh=mesh,
                       scratch_types=scratch)
    def edge_pass(*refs):
        it = iter(refs)
        vals_hbm = [next(it) for _ in range(num_vals)] if gather else []
        src_hbm = next(it) if gather else None
        dst_hbm = next(it)
        out_hbm = next(it)
        src_v = next(it) if gather else None
        dst_v = next(it)
        vals_v = [next(it) for _ in range(num_vals)]
        zb = next(it)
        stg = next(it)
        vshared = [next(it) for _ in range(num_vals)] if gather else []
        acc = [next(it) for _ in range(num_vals)]
        sem = next(it)

        cid = lax.axis_index("c")
        sid = lax.axis_index("s")
        w = cid * NS + sid

        def zstep(i, _):
            zb[pl.ds(i * 16, 16)] = jnp.zeros((16,), jnp.float32)
            return 0
        lax.fori_loop(0, ZCH // 16, zstep, 0)
        for k in range(num_vals):
            pltpu.sync_copy(zb, acc[k].at[pl.ds(sid * ZCH, ZCH)])

        if gather:
            for k in range(num_vals):
                pltpu.sync_copy(vals_hbm[k].at[pl.ds(sid * VCH, VCH)],
                                stg.at[pl.ds(0, VCH)])
                pltpu.sync_copy(stg.at[pl.ds(0, VCH)],
                                vshared[k].at[pl.ds(sid * VCH, VCH)])
        plsc.subcore_barrier()

        if gather:
            pltpu.sync_copy(src_hbm.at[w], src_v)
        pltpu.sync_copy(dst_hbm.at[w], dst_v)

        if not gather:
            def frow(i, _):
                def fcol(j, _):
                    vals_v[0][i, pl.ds(j * 16, 16)] = jnp.ones(
                        (16,), jnp.float32)
                    return 0
                lax.fori_loop(0, LANES // 16, fcol, 0)
                return 0
            lax.fori_loop(0, CHUNKS, frow, 0)

        U = 7 if num_vals == 1 else 4
        sem_g, sem_s = sem

        def group(g, _):
            base = g * U
            if gather:
                descs = [pltpu.async_copy(vshared[k].at[src_v.at[base + u]],
                                          vals_v[k].at[base + u], sem_g)
                         for u in range(U) for k in range(num_vals)]
                for d in descs:
                    d.wait()
            for u in range(U):
                for k in range(num_vals):
                    pltpu.async_copy(vals_v[k].at[base + u],
                                     acc[k].at[dst_v.at[base + u]], sem_s,
                                     add=True)
            return 0
        lax.fori_loop(0, CHUNKS // U, group, 0)

        def drain(j, _):
            for k in range(num_vals):
                pltpu.make_async_copy(vals_v[k].at[0],
                                      acc[k].at[dst_v.at[0]], sem_s).wait()
            return 0
        lax.fori_loop(0, CHUNKS, drain, 0)

        plsc.subcore_barrier()
        for k in range(num_vals):
            pltpu.sync_copy(acc[k].at[pl.ds(sid * ZCH, ZCH)], stg)
            pltpu.sync_copy(stg, out_hbm.at[cid, k, pl.ds(sid * ZCH, ZCH)])

    return edge_pass


_prop1_pass = _make_edge_pass(1, gather=True)
_k1_pass = _make_k1()
_k2_pass = _make_k2()


def _tc2(dinv, ahat, chat, accac, W1, W2, b2, W3, Wl):
    def body(dinv_ref, ahat_ref, chat_ref, acc_ref, w1_ref, w2_ref, b2_ref,
             w3_ref, wl_ref, vhat_ref):
        w1 = w1_ref[...]
        up = jnp.maximum(w1, 0.0) @ w2_ref[...]
        um = jnp.minimum(w1, 0.0) @ w2_ref[...]
        gv = w3_ref[...] @ wl_ref[...]
        b2v = b2_ref[...]
        dinv = dinv_ref[...]
        pa = dinv * (acc_ref[0, 0] + acc_ref[1, 0] + ahat_ref[...])
        pc = dinv * (acc_ref[0, 1] + acc_ref[1, 1] + chat_ref[...])
        v = jnp.zeros_like(pa)
        for j in range(H):
            v = v + jnp.maximum(pa * up[0, j] + pc * um[0, j] + b2v[0, j],
                                0.0) * gv[j, 0]
        vhat_ref[...] = dinv * v

    full = lambda s: pl.BlockSpec(s, lambda i: (0,) * len(s))
    return pl.pallas_call(
        body,
        grid=(NROW,),
        in_specs=[
            pl.BlockSpec((8, 128), lambda i: (i, 0)),
            pl.BlockSpec((8, 128), lambda i: (i, 0)),
            pl.BlockSpec((8, 128), lambda i: (i, 0)),
            pl.BlockSpec((2, 2, 8, 128), lambda i: (0, 0, i, 0)),
            full((1, H)), full((H, H)), full((1, H)), full((H, H)),
            full((H, 1)),
        ],
        out_specs=pl.BlockSpec((8, 128), lambda i: (i, 0)),
        out_shape=jax.ShapeDtypeStruct((NROW * 8, 128), jnp.float32),
    )(dinv, ahat, chat, accac, W1, W2, b2, W3, Wl)


def _tc3(dinv, vhat, accv, batch2, b3, Wl, bl):
    def body(dinv_ref, vhat_ref, acc_ref, batch_ref, b3_ref, wl_ref, bl_ref,
             out_ref):
        r = dinv_ref[...] * (acc_ref[0] + acc_ref[1] + vhat_ref[...])
        gids = lax.broadcasted_iota(jnp.int32, (G, 1), 0)
        sums = jnp.zeros((G, 1), jnp.float32)
        cnts = jnp.zeros((G, 1), jnp.float32)
        for i in range(NROW):
            oh = (batch_ref[i:i + 1, :] == gids).astype(jnp.float32)
            sums = sums + lax.dot_general(
                oh, r[i:i + 1, :], (((1,), (1,)), ((), ())))
            cnts = cnts + jnp.sum(oh, axis=1, keepdims=True)
        cst = b3_ref[...] @ wl_ref[...] + bl_ref[...]
        out_ref[...] = sums / jnp.maximum(cnts, 1.0) + cst

    return pl.pallas_call(
        body,
        out_shape=jax.ShapeDtypeStruct((G, 1), jnp.float32),
    )(dinv, vhat, accv, batch2, b3, Wl, bl)


def kernel(x, edge_index, batch, W1, b1, W2, b2, W3, b3, Wl, bl):
    src = edge_index[0]
    dst = edge_index[1]

    npad = EPAD - E
    src_p = jnp.concatenate([src, jnp.zeros((npad,), jnp.int32)])
    trash = VPAD + (jnp.arange(npad, dtype=jnp.int32) % (NACC - VPAD))
    dst_p = jnp.concatenate([dst, trash])
    src3 = src_p.reshape(NW, CHUNKS, LANES)
    dst3 = dst_p.reshape(NW, CHUNKS, LANES)

    xv = jnp.pad(x[:, 0], (0, VPAD - N))
    batch_p = jnp.pad(batch, (0, VPAD - N), constant_values=1 << 20)
    batch2 = batch_p.reshape(NROW, NCOL)

    accx, dinvc, xhatc = _k1_pass(xv, src3, dst3)

    accac, ahc, chc = _k2_pass(accx, dinvc, xhatc, src3, dst3)

    dinv2 = dinvc.reshape(NROW * 8, 128)
    accac2 = accac.reshape(2, 2, NACC)[:, :, :VPAD].reshape(
        2, 2, NROW * 8, 128)

    b2r = b2.reshape(1, H)
    vhat2 = _tc2(dinv2, ahc.reshape(NROW * 8, 128),
                 chc.reshape(NROW * 8, 128), accac2, W1, W2, b2r,
                 W3, Wl)

    accv = _prop1_pass(vhat2.reshape(VPAD), src3, dst3)
    accv2 = accv[:, 0, :VPAD].reshape(2, NROW, NCOL)

    return _tc3(dinv2.reshape(NROW, NCOL), vhat2.reshape(NROW, NCOL),
                accv2, batch2, b3.reshape(1, H), Wl, bl.reshape(1, 1))


# speedup vs baseline: 1.5388x; 1.2489x over previous
import functools

import jax
import jax.numpy as jnp
from jax import lax
from jax.experimental import pallas as pl
from jax.experimental.pallas import tpu as pltpu
from jax.experimental.pallas import tpu_sc as plsc

N = 50000
E = 800000
G = 64
H = 64

NC = 2
NS = 16
NW = NC * NS
LANES = 128

CHUNKS = 196
EPT = CHUNKS * LANES
EPAD = NW * EPT
VPAD = 50176
NACC = 51200
VCH = VPAD // NS
ZCH = NACC // NS
NROW = 49
NCOL = 1024


_MESH = dict(core_axis_name="c", subcore_axis_name="s",
             num_cores=NC, num_subcores=NS)


def _zero_fill(zb):
    def zstep(i, _):
        zb[pl.ds(i * 16, 16)] = jnp.zeros((16,), jnp.float32)
        return 0
    lax.fori_loop(0, ZCH // 16, zstep, 0)


def _edge_stream_loop(gather, num_vals, src_v, dst_v, vals_v, vshared, acc,
                      sem_g, sem_s):
    if not gather:
        def frow(i, _):
            def fcol(j, _):
                vals_v[0][i, pl.ds(j * 16, 16)] = jnp.ones((16,), jnp.float32)
                return 0
            lax.fori_loop(0, LANES // 16, fcol, 0)
            return 0
        lax.fori_loop(0, CHUNKS, frow, 0)

    U = 7 if num_vals == 1 else 4

    def group(g, _):
        base = g * U
        if gather:
            descs = [pltpu.async_copy(vshared[k].at[src_v.at[base + u]],
                                      vals_v[k].at[base + u], sem_g)
                     for u in range(U) for k in range(num_vals)]
            for d in descs:
                d.wait()
        for u in range(U):
            for k in range(num_vals):
                pltpu.async_copy(vals_v[k].at[base + u],
                                 acc[k].at[dst_v.at[base + u]], sem_s,
                                 add=True)
        return 0
    lax.fori_loop(0, CHUNKS // U, group, 0)

    def drain(j, _):
        for k in range(num_vals):
            pltpu.make_async_copy(vals_v[k].at[0],
                                  acc[k].at[dst_v.at[0]], sem_s).wait()
        return 0
    lax.fori_loop(0, CHUNKS, drain, 0)


def _newton_rsqrt(x):
    ih = jnp.int32(0x5F3759DF) - lax.shift_right_logical(
        lax.bitcast_convert_type(x, jnp.int32), 1)
    y = lax.bitcast_convert_type(ih, jnp.float32)
    for _ in range(3):
        y = y * (1.5 - 0.5 * x * y * y)
    return y


def _ones_fill(vals_v):
    def frow(i, _):
        def fcol(j, _):
            vals_v[i, pl.ds(j * 16, 16)] = jnp.ones((16,), jnp.float32)
            return 0
        lax.fori_loop(0, LANES // 16, fcol, 0)
        return 0
    lax.fori_loop(0, CHUNKS, frow, 0)


def _make_k1():
    mesh = plsc.VectorSubcoreMesh(**_MESH)
    scratch = [
        pltpu.VMEM((VCH,), jnp.float32),
        pltpu.VMEM((VCH,), jnp.float32),
        pltpu.VMEM((VCH,), jnp.float32),
        pltpu.VMEM((CHUNKS, LANES), jnp.int32),
        pltpu.VMEM((CHUNKS, LANES), jnp.int32),
        pltpu.VMEM((CHUNKS, LANES), jnp.float32),
        pltpu.VMEM((ZCH,), jnp.float32),
        pltpu.VMEM_SHARED((VPAD,), jnp.float32),
        pltpu.VMEM_SHARED((NACC,), jnp.float32),
        pltpu.VMEM_SHARED((NACC,), jnp.float32),
        (pltpu.SemaphoreType.DMA, pltpu.SemaphoreType.DMA),
    ]
    out_type = (jax.ShapeDtypeStruct((NC * NACC,), jnp.float32),
                jax.ShapeDtypeStruct((VPAD,), jnp.float32),
                jax.ShapeDtypeStruct((VPAD,), jnp.float32))

    @functools.partial(pl.kernel, out_type=out_type, mesh=mesh,
                       scratch_types=scratch)
    def k1(x_hbm, src_hbm, dst_hbm, accx_hbm, dinv_hbm, xhat_hbm,
           d0b, d1b, xb, src_v, dst_v, vals_v, zb, vsh, dacc, acc, sems):
        sem_g, sem_s = sems
        cid = lax.axis_index("c")
        sid = lax.axis_index("s")
        w = cid * NS + sid
        sl = pl.ds(sid * VCH, VCH)

        _zero_fill(zb)
        pltpu.sync_copy(zb, dacc.at[pl.ds(sid * ZCH, ZCH)])
        pltpu.sync_copy(zb, acc.at[pl.ds(sid * ZCH, ZCH)])
        _ones_fill(vals_v)
        pltpu.sync_copy(x_hbm.at[sl], xb)
        plsc.subcore_barrier()

        pltpu.sync_copy(dst_hbm.at[2 * sid], src_v)
        pltpu.sync_copy(dst_hbm.at[2 * sid + 1], dst_v)

        def dgroup(g, _):
            base = g * 7
            for u in range(7):
                pltpu.async_copy(vals_v.at[base + u],
                                 dacc.at[src_v.at[base + u]], sem_s,
                                 add=True)
                pltpu.async_copy(vals_v.at[base + u],
                                 dacc.at[dst_v.at[base + u]], sem_s,
                                 add=True)
            return 0
        lax.fori_loop(0, CHUNKS // 7, dgroup, 0)

        def ddrain(j, _):
            pltpu.make_async_copy(vals_v.at[0], dacc.at[src_v.at[0]],
                                  sem_s).wait()
            return 0
        lax.fori_loop(0, 2 * CHUNKS, ddrain, 0)
        plsc.subcore_barrier()

        pltpu.sync_copy(dacc.at[sl], d0b)

        def dstep(i, _):
            v = pl.ds(i * 16, 16)
            deg = d0b[v] + 1.0
            y = _newton_rsqrt(deg)
            d0b[v] = y
            d1b[v] = y * xb[v]
            return 0
        lax.fori_loop(0, VCH // 16, dstep, 0)

        pltpu.sync_copy(d0b, dinv_hbm.at[sl])
        pltpu.sync_copy(d1b, xhat_hbm.at[sl])
        pltpu.sync_copy(d1b, vsh.at[sl])
        plsc.subcore_barrier()

        pltpu.sync_copy(src_hbm.at[w], src_v)
        pltpu.sync_copy(dst_hbm.at[w], dst_v)
        _edge_stream_loop(True, 1, src_v, dst_v, [vals_v], [vsh], [acc],
                          sem_g, sem_s)

        plsc.subcore_barrier()
        pltpu.sync_copy(acc.at[pl.ds(sid * ZCH, ZCH)], zb)
        pltpu.sync_copy(zb, accx_hbm.at[pl.ds(cid * NACC + sid * ZCH, ZCH)])

    return k1


def _make_k2():
    mesh = plsc.VectorSubcoreMesh(**_MESH)
    scratch = [
        pltpu.VMEM((VCH,), jnp.float32),
        pltpu.VMEM((VCH,), jnp.float32),
        pltpu.VMEM((VCH,), jnp.float32),
        pltpu.VMEM((VCH,), jnp.float32),
        pltpu.VMEM((CHUNKS, LANES), jnp.int32),
        pltpu.VMEM((CHUNKS, LANES), jnp.int32),
        pltpu.VMEM((CHUNKS, LANES), jnp.float32),
        pltpu.VMEM((CHUNKS, LANES), jnp.float32),
        pltpu.VMEM((ZCH,), jnp.float32),
        pltpu.VMEM_SHARED((VPAD,), jnp.float32),
        pltpu.VMEM_SHARED((VPAD,), jnp.float32),
        pltpu.VMEM_SHARED((NACC,), jnp.float32),
        pltpu.VMEM_SHARED((NACC,), jnp.float32),
        (pltpu.SemaphoreType.DMA, pltpu.SemaphoreType.DMA),
    ]
    out_type = (jax.ShapeDtypeStruct((NC * 2 * NACC,), jnp.float32),
                jax.ShapeDtypeStruct((VPAD,), jnp.float32),
                jax.ShapeDtypeStruct((VPAD,), jnp.float32))

    @functools.partial(pl.kernel, out_type=out_type, mesh=mesh,
                       scratch_types=scratch)
    def k2(accx_hbm, dinv_hbm, xhat_hbm, src_hbm, dst_hbm,
           accac_hbm, ah_hbm, ch_hbm,
           a0b, a1b, xb, db, src_v, dst_v, va, vc, zb,
           vsh0, vsh1, acc0, acc1, sems):
        sem_g, sem_s = sems
        cid = lax.axis_index("c")
        sid = lax.axis_index("s")
        w = cid * NS + sid
        sl = pl.ds(sid * VCH, VCH)

        _zero_fill(zb)
        pltpu.sync_copy(zb, acc0.at[pl.ds(sid * ZCH, ZCH)])
        pltpu.sync_copy(zb, acc1.at[pl.ds(sid * ZCH, ZCH)])

        pltpu.sync_copy(accx_hbm.at[pl.ds(sid * VCH, VCH)], a0b)
        pltpu.sync_copy(accx_hbm.at[pl.ds(NACC + sid * VCH, VCH)], a1b)
        pltpu.sync_copy(xhat_hbm.at[sl], xb)
        pltpu.sync_copy(dinv_hbm.at[sl], db)

        def dstep(i, _):
            v = pl.ds(i * 16, 16)
            dv = db[v]
            s = dv * (a0b[v] + a1b[v] + xb[v])
            a0b[v] = dv * jnp.maximum(s, 0.0)
            a1b[v] = dv * jnp.minimum(s, 0.0)
            return 0
        lax.fori_loop(0, VCH // 16, dstep, 0)

        pltpu.sync_copy(a0b, ah_hbm.at[sl])
        pltpu.sync_copy(a1b, ch_hbm.at[sl])
        pltpu.sync_copy(a0b, vsh0.at[sl])
        pltpu.sync_copy(a1b, vsh1.at[sl])
        plsc.subcore_barrier()

        pltpu.sync_copy(src_hbm.at[w], src_v)
        pltpu.sync_copy(dst_hbm.at[w], dst_v)
        _edge_stream_loop(True, 2, src_v, dst_v, [va, vc], [vsh0, vsh1],
                          [acc0, acc1], sem_g, sem_s)

        plsc.subcore_barrier()
        for k, a in enumerate((acc0, acc1)):
            pltpu.sync_copy(a.at[pl.ds(sid * ZCH, ZCH)], zb)
            pltpu.sync_copy(
                zb,
                accac_hbm.at[pl.ds((cid * 2 + k) * NACC + sid * ZCH, ZCH)])

    return k2


def _make_edge_pass(num_vals, gather):
    mesh = plsc.VectorSubcoreMesh(
        core_axis_name="c", subcore_axis_name="s",
        num_cores=NC, num_subcores=NS)

    scratch = []
    if gather:
        scratch.append(pltpu.VMEM((CHUNKS, LANES), jnp.int32))
    scratch.append(pltpu.VMEM((CHUNKS, LANES), jnp.int32))
    for _ in range(num_vals):
        scratch.append(pltpu.VMEM((CHUNKS, LANES), jnp.float32))
    scratch.append(pltpu.VMEM((ZCH,), jnp.float32))
    scratch.append(pltpu.VMEM((ZCH,), jnp.float32))
    if gather:
        for _ in range(num_vals):
            scratch.append(pltpu.VMEM_SHARED((VPAD,), jnp.float32))
    for _ in range(num_vals):
        scratch.append(pltpu.VMEM_SHARED((NACC,), jnp.float32))
    scratch.append((pltpu.SemaphoreType.DMA, pltpu.SemaphoreType.DMA))

    out_type = jax.ShapeDtypeStruct((NC, num_vals, NACC), jnp.float32)

    @functools.partial(pl.kernel, out_type=out_type, mesh=mesh,
                       scratch_types=scratch)
    def edge_pass(*refs):
        it = iter(refs)
        vals_hbm = [next(it) for _ in range(num_vals)] if gather else []
        src_hbm = next(it) if gather else None
        dst_hbm = next(it)
        out_hbm = next(it)
        src_v = next(it) if gather else None
        dst_v = next(it)
        vals_v = [next(it) for _ in range(num_vals)]
        zb = next(it)
        stg = next(it)
        vshared = [next(it) for _ in range(num_vals)] if gather else []
        acc = [next(it) for _ in range(num_vals)]
        sem = next(it)

        cid = lax.axis_index("c")
        sid = lax.axis_index("s")
        w = cid * NS + sid

        def zstep(i, _):
            zb[pl.ds(i * 16, 16)] = jnp.zeros((16,), jnp.float32)
            return 0
        lax.fori_loop(0, ZCH // 16, zstep, 0)
        for k in range(num_vals):
            pltpu.sync_copy(zb, acc[k].at[pl.ds(sid * ZCH, ZCH)])

        if gather:
            for k in range(num_vals):
                pltpu.sync_copy(vals_hbm[k].at[pl.ds(sid * VCH, VCH)],
                                stg.at[pl.ds(0, VCH)])
                pltpu.sync_copy(stg.at[pl.ds(0, VCH)],
                                vshared[k].at[pl.ds(sid * VCH, VCH)])
        plsc.subcore_barrier()

        if gather:
            pltpu.sync_copy(src_hbm.at[w], src_v)
        pltpu.sync_copy(dst_hbm.at[w], dst_v)

        if not gather:
            def frow(i, _):
                def fcol(j, _):
                    vals_v[0][i, pl.ds(j * 16, 16)] = jnp.ones(
                        (16,), jnp.float32)
                    return 0
                lax.fori_loop(0, LANES // 16, fcol, 0)
                return 0
            lax.fori_loop(0, CHUNKS, frow, 0)

        U = 7 if num_vals == 1 else 4
        sem_g, sem_s = sem

        def group(g, _):
            base = g * U
            if gather:
                descs = [pltpu.async_copy(vshared[k].at[src_v.at[base + u]],
                                          vals_v[k].at[base + u], sem_g)
                         for u in range(U) for k in range(num_vals)]
                for d in descs:
                    d.wait()
            for u in range(U):
                for k in range(num_vals):
                    pltpu.async_copy(vals_v[k].at[base + u],
                                     acc[k].at[dst_v.at[base + u]], sem_s,
                                     add=True)
            return 0
        lax.fori_loop(0, CHUNKS // U, group, 0)

        def drain(j, _):
            for k in range(num_vals):
                pltpu.make_async_copy(vals_v[k].at[0],
                                      acc[k].at[dst_v.at[0]], sem_s).wait()
            return 0
        lax.fori_loop(0, CHUNKS, drain, 0)

        plsc.subcore_barrier()
        for k in range(num_vals):
            pltpu.sync_copy(acc[k].at[pl.ds(sid * ZCH, ZCH)], stg)
            pltpu.sync_copy(stg, out_hbm.at[cid, k, pl.ds(sid * ZCH, ZCH)])

    return edge_pass


_prop1_pass = _make_edge_pass(1, gather=True)
_k1_pass = _make_k1()
_k2_pass = _make_k2()


def _tc2(dinv, ahat, chat, accac, W1, W2, b2, W3, Wl):
    def body(dinv_ref, ahat_ref, chat_ref, acc_ref, w1_ref, w2_ref, b2_ref,
             w3_ref, wl_ref, vhat_ref):
        w1 = w1_ref[...]
        up = jnp.maximum(w1, 0.0) @ w2_ref[...]
        um = jnp.minimum(w1, 0.0) @ w2_ref[...]
        gv = w3_ref[...] @ wl_ref[...]
        b2v = b2_ref[...]
        dinv = dinv_ref[...]
        pa = dinv * (acc_ref[0, 0] + acc_ref[1, 0] + ahat_ref[...])
        pc = dinv * (acc_ref[0, 1] + acc_ref[1, 1] + chat_ref[...])
        v = jnp.zeros_like(pa)
        for j in range(H):
            v = v + jnp.maximum(pa * up[0, j] + pc * um[0, j] + b2v[0, j],
                                0.0) * gv[j, 0]
        vhat_ref[...] = dinv * v

    full = lambda s: pl.BlockSpec(s, lambda i: (0,) * len(s))
    return pl.pallas_call(
        body,
        grid=(NROW * 8 // 56,),
        in_specs=[
            pl.BlockSpec((56, 128), lambda i: (i, 0)),
            pl.BlockSpec((56, 128), lambda i: (i, 0)),
            pl.BlockSpec((56, 128), lambda i: (i, 0)),
            pl.BlockSpec((2, 2, 56, 128), lambda i: (0, 0, i, 0)),
            full((1, H)), full((H, H)), full((1, H)), full((H, H)),
            full((H, 1)),
        ],
        out_specs=pl.BlockSpec((56, 128), lambda i: (i, 0)),
        out_shape=jax.ShapeDtypeStruct((NROW * 8, 128), jnp.float32),
    )(dinv, ahat, chat, accac, W1, W2, b2, W3, Wl)


def _tc3(dinv, vhat, accv, batch2, b3, Wl, bl):
    def body(dinv_ref, vhat_ref, acc_ref, batch_ref, b3_ref, wl_ref, bl_ref,
             out_ref):
        r = dinv_ref[...] * (acc_ref[0] + acc_ref[1] + vhat_ref[...])
        gids = lax.broadcasted_iota(jnp.int32, (G, 1), 0)
        sums = jnp.zeros((G, 1), jnp.float32)
        cnts = jnp.zeros((G, 1), jnp.float32)
        for i in range(NROW):
            oh = (batch_ref[i:i + 1, :] == gids).astype(jnp.float32)
            sums = sums + lax.dot_general(
                oh, r[i:i + 1, :], (((1,), (1,)), ((), ())))
            cnts = cnts + jnp.sum(oh, axis=1, keepdims=True)
        cst = b3_ref[...] @ wl_ref[...] + bl_ref[...]
        out_ref[...] = sums / jnp.maximum(cnts, 1.0) + cst

    return pl.pallas_call(
        body,
        out_shape=jax.ShapeDtypeStruct((G, 1), jnp.float32),
    )(dinv, vhat, accv, batch2, b3, Wl, bl)


def _edge_prep(edge_index):
    def body(ei_ref, src_ref, dst_ref):
        i = pl.program_id(0)
        blk = ei_ref[...]
        sv = blk[0:1, :].reshape(CHUNKS, LANES)
        dv = blk[1:2, :].reshape(CHUNKS, LANES)
        rows = lax.broadcasted_iota(jnp.int32, (CHUNKS, LANES), 0) + i * CHUNKS
        cols = lax.broadcasted_iota(jnp.int32, (CHUNKS, LANES), 1)
        valid = rows < (E // LANES)
        trash = VPAD + ((rows * LANES + cols) % (NACC - VPAD))
        src_ref[0] = jnp.where(valid, sv, 0)
        dst_ref[0] = jnp.where(valid, dv, trash)

    return pl.pallas_call(
        body,
        grid=(NW,),
        in_specs=[pl.BlockSpec((2, EPT), lambda i: (0, i))],
        out_specs=(pl.BlockSpec((1, CHUNKS, LANES), lambda i: (i, 0, 0)),
                   pl.BlockSpec((1, CHUNKS, LANES), lambda i: (i, 0, 0))),
        out_shape=(jax.ShapeDtypeStruct((NW, CHUNKS, LANES), jnp.int32),
                   jax.ShapeDtypeStruct((NW, CHUNKS, LANES), jnp.int32)),
    )(edge_index)


def kernel(x, edge_index, batch, W1, b1, W2, b2, W3, b3, Wl, bl):
    src3, dst3 = _edge_prep(edge_index)

    xv = jnp.pad(x[:, 0], (0, VPAD - N))
    batch_p = jnp.pad(batch, (0, VPAD - N), constant_values=1 << 20)
    batch2 = batch_p.reshape(NROW, NCOL)

    accx, dinvc, xhatc = _k1_pass(xv, src3, dst3)

    accac, ahc, chc = _k2_pass(accx, dinvc, xhatc, src3, dst3)

    dinv2 = dinvc.reshape(NROW * 8, 128)
    accac2 = accac.reshape(2, 2, NACC)[:, :, :VPAD].reshape(
        2, 2, NROW * 8, 128)

    b2r = b2.reshape(1, H)
    vhat2 = _tc2(dinv2, ahc.reshape(NROW * 8, 128),
                 chc.reshape(NROW * 8, 128), accac2, W1, W2, b2r,
                 W3, Wl)

    accv = _prop1_pass(vhat2.reshape(VPAD), src3, dst3)
    accv2 = accv[:, 0, :VPAD].reshape(2, NROW, NCOL)

    return _tc3(dinv2.reshape(NROW, NCOL), vhat2.reshape(NROW, NCOL),
                accv2, batch2, b3.reshape(1, H), Wl, bl.reshape(1, 1))
